# baseline (speedup 1.0000x reference)
import functools

import jax
import jax.numpy as jnp
from jax import lax
from jax.experimental import pallas as pl
from jax.experimental.pallas import tpu as pltpu
from jax.experimental.pallas import tpu_sc as plsc

BATCH = 4
SEQ_LEN = 2048
EMBED_DIM = 768
LANES = 16

NUM_CORES = 2
NUM_SUBCORES = 16
NUM_WORKERS = NUM_CORES * NUM_SUBCORES

S_PER_W = SEQ_LEN // NUM_WORKERS
SUB = 8
NSTEP = S_PER_W // SUB
ROWS = BATCH * SUB
NGRP = 4
AHEAD = 2
COLS = EMBED_DIM // LANES
N_ROWS = BATCH * SEQ_LEN
NIDX = BATCH * S_PER_W

_mesh = plsc.VectorSubcoreMesh(core_axis_name="c", subcore_axis_name="s")

_scratch = (
    [pltpu.VMEM((NIDX,), jnp.int32)]
    + [pltpu.VMEM((ROWS, EMBED_DIM), jnp.float32) for _ in range(NGRP)]
    + [pltpu.VMEM((SUB, EMBED_DIM), jnp.float32) for _ in range(NGRP)]
    + [pltpu.SemaphoreType.DMA for _ in range(1 + 3 * NGRP)]
)


@functools.partial(
    pl.kernel,
    mesh=_mesh,
    out_type=jax.ShapeDtypeStruct((N_ROWS, EMBED_DIM), jnp.float32),
    scratch_types=_scratch,
)
def _embed_kernel(ids_hbm, tok_hbm, pos_hbm, out_hbm, idx_v, *rest):
    tbufs = rest[:NGRP]
    pbufs = rest[NGRP:2 * NGRP]
    sems = rest[2 * NGRP:]
    sem_idx = sems[0]
    gsems = sems[1:1 + NGRP]
    psems = sems[1 + NGRP:1 + 2 * NGRP]
    ssems = sems[1 + 2 * NGRP:1 + 3 * NGRP]

    wid = lax.axis_index("s") * NUM_CORES + lax.axis_index("c")
    s0 = wid * S_PER_W

    cp_idx = pltpu.async_copy(ids_hbm.at[pl.ds(wid * NIDX, NIDX)], idx_v,
                              sem_idx)
    cp_idx.wait()

    def fetch(t):
        g = t % NGRP
        gcp = pltpu.async_copy(
            tok_hbm.at[idx_v.at[pl.ds(t * ROWS, ROWS)]], tbufs[g], gsems[g])
        pcp = pltpu.async_copy(
            pos_hbm.at[pl.ds(s0 + t * SUB, SUB)], pbufs[g], psems[g])
        return gcp, pcp

    def stores(t):
        g = t % NGRP
        return [pltpu.async_copy(
            tbufs[g].at[pl.ds(b * SUB, SUB)],
            out_hbm.at[pl.ds(b * SEQ_LEN + s0 + t * SUB, SUB)],
            ssems[g]) for b in range(BATCH)]

    def add_pos(t):
        g = t % NGRP
        buf = tbufs[g]
        pbuf = pbufs[g]

        @plsc.parallel_loop(0, SUB)
        def _row(r):
            for j in range(COLS):
                sl = pl.ds(j * LANES, LANES)
                pvec = pbuf[r, sl]
                for b in range(BATCH):
                    plsc.addupdate(buf.at[b * SUB + r, sl], pvec)

    fcp = [None] * NGRP
    scp = [None] * NGRP
    for t in range(AHEAD):
        fcp[t] = fetch(t)
    for t in range(NSTEP):
        g = t % NGRP
        if t + AHEAD < NSTEP:
            ag = (t + AHEAD) % NGRP
            if scp[ag] is not None:
                for c in scp[ag]:
                    c.wait()
            fcp[ag] = fetch(t + AHEAD)
        fcp[g][0].wait()
        fcp[g][1].wait()
        add_pos(t)
        scp[g] = stores(t)
    for p in range(NGRP):
        if scp[p] is not None:
            for c in scp[p]:
                c.wait()


def kernel(input_ids, token_table, position_table):
    ids_re = (input_ids.astype(jnp.int32)
              .reshape(BATCH, NUM_WORKERS, NSTEP, SUB)
              .transpose(1, 2, 0, 3)
              .reshape(N_ROWS))
    out = _embed_kernel(ids_re, token_table, position_table)
    return out.reshape(BATCH, SEQ_LEN, EMBED_DIM)


# speedup vs baseline: 1.0276x; 1.0276x over previous
import functools

import jax
import jax.numpy as jnp
from jax import lax
from jax.experimental import pallas as pl
from jax.experimental.pallas import tpu as pltpu
from jax.experimental.pallas import tpu_sc as plsc

BATCH = 4
SEQ_LEN = 2048
EMBED_DIM = 768
LANES = 16

NUM_CORES = 2
NUM_SUBCORES = 16
NUM_WORKERS = NUM_CORES * NUM_SUBCORES

S_PER_W = SEQ_LEN // NUM_WORKERS
SUB = 16
NSTEP = S_PER_W // SUB
ROWS = BATCH * SUB
NGRP = 2
AHEAD = 1
COLS = EMBED_DIM // LANES
N_ROWS = BATCH * SEQ_LEN
NIDX = BATCH * S_PER_W

_mesh = plsc.VectorSubcoreMesh(core_axis_name="c", subcore_axis_name="s")

_scratch = (
    [pltpu.VMEM((NIDX,), jnp.int32)]
    + [pltpu.VMEM((ROWS, EMBED_DIM), jnp.float32) for _ in range(NGRP)]
    + [pltpu.VMEM((SUB, EMBED_DIM), jnp.float32) for _ in range(NGRP)]
    + [pltpu.SemaphoreType.DMA for _ in range(1 + 3 * NGRP)]
)


@functools.partial(
    pl.kernel,
    mesh=_mesh,
    out_type=jax.ShapeDtypeStruct((N_ROWS, EMBED_DIM), jnp.float32),
    scratch_types=_scratch,
)
def _embed_kernel(ids_hbm, tok_hbm, pos_hbm, out_hbm, idx_v, *rest):
    tbufs = rest[:NGRP]
    pbufs = rest[NGRP:2 * NGRP]
    sems = rest[2 * NGRP:]
    sem_idx = sems[0]
    gsems = sems[1:1 + NGRP]
    psems = sems[1 + NGRP:1 + 2 * NGRP]
    ssems = sems[1 + 2 * NGRP:1 + 3 * NGRP]

    wid = lax.axis_index("s") * NUM_CORES + lax.axis_index("c")
    s0 = wid * S_PER_W

    cp_idx = pltpu.async_copy(ids_hbm.at[pl.ds(wid * NIDX, NIDX)], idx_v,
                              sem_idx)
    cp_idx.wait()

    def fetch(t):
        g = t % NGRP
        gcp = pltpu.async_copy(
            tok_hbm.at[idx_v.at[pl.ds(t * ROWS, ROWS)]], tbufs[g], gsems[g])
        pcp = pltpu.async_copy(
            pos_hbm.at[pl.ds(s0 + t * SUB, SUB)], pbufs[g], psems[g])
        return gcp, pcp

    def stores(t):
        g = t % NGRP
        return [pltpu.async_copy(
            tbufs[g].at[pl.ds(b * SUB, SUB)],
            out_hbm.at[pl.ds(b * SEQ_LEN + s0 + t * SUB, SUB)],
            ssems[g]) for b in range(BATCH)]

    def add_pos(t):
        g = t % NGRP
        buf = tbufs[g]
        pbuf = pbufs[g]

        @plsc.parallel_loop(0, SUB)
        def _row(r):
            for j in range(COLS):
                sl = pl.ds(j * LANES, LANES)
                pvec = pbuf[r, sl]
                for b in range(BATCH):
                    plsc.addupdate(buf.at[b * SUB + r, sl], pvec)

    fcp = [None] * NGRP
    scp = [None] * NGRP
    for t in range(AHEAD):
        fcp[t] = fetch(t)
    for t in range(NSTEP):
        g = t % NGRP
        if t + AHEAD < NSTEP:
            ag = (t + AHEAD) % NGRP
            if scp[ag] is not None:
                for c in scp[ag]:
                    c.wait()
            fcp[ag] = fetch(t + AHEAD)
        fcp[g][0].wait()
        fcp[g][1].wait()
        add_pos(t)
        scp[g] = stores(t)
    for p in range(NGRP):
        if scp[p] is not None:
            for c in scp[p]:
                c.wait()


def kernel(input_ids, token_table, position_table):
    ids_re = (input_ids.astype(jnp.int32)
              .reshape(BATCH, NUM_WORKERS, NSTEP, SUB)
              .transpose(1, 2, 0, 3)
              .reshape(N_ROWS))
    out = _embed_kernel(ids_re, token_table, position_table)
    return out.reshape(BATCH, SEQ_LEN, EMBED_DIM)
